# baseline (speedup 1.0000x reference)
import functools

import jax
import jax.numpy as jnp
from jax import lax
from jax.experimental import pallas as pl
from jax.experimental.pallas import tpu as pltpu
from jax.experimental.pallas import tpu_sc as plsc

_CHUNK = 32
_NBUF = 4


def _make_sc_gather(V, D, B, S, per_tile, num_cores, num_subcores):
    mesh = plsc.VectorSubcoreMesh(core_axis_name="c", subcore_axis_name="s")
    n_chunks = per_tile // _CHUNK
    N = B * S

    @functools.partial(
        pl.kernel,
        out_type=jax.ShapeDtypeStruct((N, D), jnp.float32),
        mesh=mesh,
        compiler_params=pltpu.CompilerParams(needs_layout_passes=False),
        scratch_types=[
            pltpu.VMEM((per_tile,), jnp.int32),
            *[pltpu.VMEM((_CHUNK, D), jnp.float32) for _ in range(_NBUF)],
            *[pltpu.SemaphoreType.DMA for _ in range(2 * _NBUF)],
        ],
    )
    def sc_gather(w_hbm, x_hbm, out_hbm, idx_v, *rest):
        bufs = rest[:_NBUF]
        gsems = rest[_NBUF:2 * _NBUF]
        osems = rest[2 * _NBUF:]
        wid = lax.axis_index("s") * num_cores + lax.axis_index("c")
        base = wid * per_tile
        pltpu.sync_copy(x_hbm.at[pl.ds(base, per_tile)], idx_v)

        def g(c):
            return pltpu.make_async_copy(
                w_hbm.at[idx_v.at[pl.ds(c * _CHUNK, _CHUNK)]],
                bufs[c % _NBUF], gsems[c % _NBUF])

        def o(c):
            return pltpu.make_async_copy(
                bufs[c % _NBUF],
                out_hbm.at[pl.ds(base + c * _CHUNK, _CHUNK)],
                osems[c % _NBUF])

        for c in range(min(_NBUF, n_chunks)):
            g(c).start()
        for c in range(n_chunks):
            g(c).wait()
            o(c).start()
            if c + _NBUF < n_chunks:
                o(c).wait()
                g(c + _NBUF).start()
        for c in range(max(0, n_chunks - _NBUF), n_chunks):
            o(c).wait()

    return sc_gather


def kernel(x, W_E):
    B, S = x.shape
    D, V = W_E.shape
    N = B * S
    info = plsc.get_sparse_core_info()
    num_tiles = info.num_cores * info.num_subcores
    per_tile = N // num_tiles
    assert N % num_tiles == 0 and per_tile % _CHUNK == 0
    assert S % per_tile == 0

    W_T = W_E.T
    xi = x.reshape(N).astype(jnp.int32)
    out = _make_sc_gather(V, D, B, S, per_tile, info.num_cores,
                          info.num_subcores)(W_T, xi)
    return out.reshape(B, S, D)


# speedup vs baseline: 1.0010x; 1.0010x over previous
import functools

import jax
import jax.numpy as jnp
from jax import lax
from jax.experimental import pallas as pl
from jax.experimental.pallas import tpu as pltpu
from jax.experimental.pallas import tpu_sc as plsc

_CHUNK = 16
_NBUF = 8


def _make_sc_gather(V, D, B, S, per_tile, num_cores, num_subcores):
    mesh = plsc.VectorSubcoreMesh(core_axis_name="c", subcore_axis_name="s")
    n_chunks = per_tile // _CHUNK
    N = B * S

    @functools.partial(
        pl.kernel,
        out_type=jax.ShapeDtypeStruct((N, D), jnp.float32),
        mesh=mesh,
        compiler_params=pltpu.CompilerParams(needs_layout_passes=False),
        scratch_types=[
            pltpu.VMEM((per_tile,), jnp.int32),
            *[pltpu.VMEM((_CHUNK, D), jnp.float32) for _ in range(_NBUF)],
            *[pltpu.SemaphoreType.DMA for _ in range(2 * _NBUF)],
        ],
    )
    def sc_gather(w_hbm, x_hbm, out_hbm, idx_v, *rest):
        bufs = rest[:_NBUF]
        gsems = rest[_NBUF:2 * _NBUF]
        osems = rest[2 * _NBUF:]
        wid = lax.axis_index("s") * num_cores + lax.axis_index("c")
        base = wid * per_tile
        pltpu.sync_copy(x_hbm.at[pl.ds(base, per_tile)], idx_v)

        def g(c):
            return pltpu.make_async_copy(
                w_hbm.at[idx_v.at[pl.ds(c * _CHUNK, _CHUNK)]],
                bufs[c % _NBUF], gsems[c % _NBUF])

        def o(c):
            return pltpu.make_async_copy(
                bufs[c % _NBUF],
                out_hbm.at[pl.ds(base + c * _CHUNK, _CHUNK)],
                osems[c % _NBUF])

        for c in range(min(_NBUF, n_chunks)):
            g(c).start()
        for c in range(n_chunks):
            g(c).wait()
            o(c).start()
            if c + _NBUF < n_chunks:
                o(c).wait()
                g(c + _NBUF).start()
        for c in range(max(0, n_chunks - _NBUF), n_chunks):
            o(c).wait()

    return sc_gather


def kernel(x, W_E):
    B, S = x.shape
    D, V = W_E.shape
    N = B * S
    info = plsc.get_sparse_core_info()
    num_tiles = info.num_cores * info.num_subcores
    per_tile = N // num_tiles
    assert N % num_tiles == 0 and per_tile % _CHUNK == 0
    assert S % per_tile == 0

    W_T = W_E.T
    xi = x.reshape(N).astype(jnp.int32)
    out = _make_sc_gather(V, D, B, S, per_tile, info.num_cores,
                          info.num_subcores)(W_T, xi)
    return out.reshape(B, S, D)


# speedup vs baseline: 1.0039x; 1.0029x over previous
import functools

import jax
import jax.numpy as jnp
from jax import lax
from jax.experimental import pallas as pl
from jax.experimental.pallas import tpu as pltpu
from jax.experimental.pallas import tpu_sc as plsc

_CHUNK = 32
_NBUF = 5


def _make_sc_gather(V, D, B, S, per_tile, num_cores, num_subcores):
    mesh = plsc.VectorSubcoreMesh(core_axis_name="c", subcore_axis_name="s")
    n_chunks = per_tile // _CHUNK
    N = B * S

    @functools.partial(
        pl.kernel,
        out_type=jax.ShapeDtypeStruct((N, D), jnp.float32),
        mesh=mesh,
        compiler_params=pltpu.CompilerParams(needs_layout_passes=False),
        scratch_types=[
            pltpu.VMEM((per_tile,), jnp.int32),
            *[pltpu.VMEM((_CHUNK, D), jnp.float32) for _ in range(_NBUF)],
            *[pltpu.SemaphoreType.DMA for _ in range(2 * _NBUF)],
        ],
    )
    def sc_gather(w_hbm, x_hbm, out_hbm, idx_v, *rest):
        bufs = rest[:_NBUF]
        gsems = rest[_NBUF:2 * _NBUF]
        osems = rest[2 * _NBUF:]
        wid = lax.axis_index("s") * num_cores + lax.axis_index("c")
        base = wid * per_tile
        pltpu.sync_copy(x_hbm.at[pl.ds(base, per_tile)], idx_v)

        def g(c):
            return pltpu.make_async_copy(
                w_hbm.at[idx_v.at[pl.ds(c * _CHUNK, _CHUNK)]],
                bufs[c % _NBUF], gsems[c % _NBUF])

        def o(c):
            return pltpu.make_async_copy(
                bufs[c % _NBUF],
                out_hbm.at[pl.ds(base + c * _CHUNK, _CHUNK)],
                osems[c % _NBUF])

        for c in range(min(_NBUF, n_chunks)):
            g(c).start()
        for c in range(n_chunks):
            g(c).wait()
            o(c).start()
            if c + _NBUF < n_chunks:
                o(c).wait()
                g(c + _NBUF).start()
        for c in range(max(0, n_chunks - _NBUF), n_chunks):
            o(c).wait()

    return sc_gather


def kernel(x, W_E):
    B, S = x.shape
    D, V = W_E.shape
    N = B * S
    info = plsc.get_sparse_core_info()
    num_tiles = info.num_cores * info.num_subcores
    per_tile = N // num_tiles
    assert N % num_tiles == 0 and per_tile % _CHUNK == 0
    assert S % per_tile == 0

    W_T = W_E.T
    xi = x.reshape(N).astype(jnp.int32)
    out = _make_sc_gather(V, D, B, S, per_tile, info.num_cores,
                          info.num_subcores)(W_T, xi)
    return out.reshape(B, S, D)
